# baseline (speedup 1.0000x reference)
import functools
import jax
import jax.numpy as jnp
import numpy as np
from jax import lax
from jax.experimental import pallas as pl
from jax.experimental.pallas import tpu as pltpu
from jax.experimental.pallas import tpu_sc as plsc

_B, _C, _L = 16, 512, 2048
_S = 4
_LS = _L // _S
_K = _LS
_BLK = 128

_DEF = jax.lax.Precision.DEFAULT


def _dot(a, b):
    return jax.lax.dot_general(a, b, (((1,), (0,)), ((), ())),
                               precision=_DEF,
                               preferred_element_type=jnp.float32)


def _front_body(x_ref, t_ref, wflat_ref, bconv_ref, cbt_ref, c2_ref,
                xde_ref, idx_ref):
    T = t_ref[...]
    xr = x_ref[0].astype(jnp.bfloat16)
    ys = [_dot(xr[:, 512 * m:512 * (m + 1)], T) for m in range(4)]
    xks = [jnp.concatenate([ys[m][:, k * _BLK:(k + 1) * _BLK]
                            for m in range(4)], axis=1) for k in range(_S)]
    xcol = jnp.concatenate(xks, axis=0)
    x_de = _dot(wflat_ref[...], xcol) + bconv_ref[...]
    x2 = jnp.sum(x_de * x_de, axis=1, keepdims=True)
    scores = _dot(x_de, cbt_ref[...])
    d2 = x2 + c2_ref[...] - 2.0 * scores
    m = jnp.min(d2, axis=1, keepdims=True)
    iota = jax.lax.broadcasted_iota(jnp.int32, (_C, _K), 1)
    idx = jnp.min(jnp.where(d2 == m, iota, _K), axis=1, keepdims=True)
    xde_ref[0] = x_de
    idx_ref[0] = idx


def _back_body(xde_ref, q_ref, w1_ref, b1_ref, w2_ref, b2_ref, out_ref):
    x_de = xde_ref[0]
    q = q_ref[0]
    t = x_de - q
    h = jnp.maximum(_dot(w1_ref[...], t) + b1_ref[...], 0.0)
    z = _dot(w2_ref[...], h) + b2_ref[...]
    out_ref[0] = z + q


def _sc_gather(table, idx_flat):
    info = plsc.get_sparse_core_info()
    NC, NS = info.num_cores, info.num_subcores
    NW = NC * NS
    Btot = idx_flat.shape[0]
    b_per_w = Btot // NW
    mesh = plsc.VectorSubcoreMesh(core_axis_name="c", subcore_axis_name="s")

    CH = 16
    @functools.partial(
        pl.kernel, mesh=mesh,
        out_type=jax.ShapeDtypeStruct((Btot, _LS), jnp.float32),
        scratch_types=[
            pltpu.VMEM((CH,), jnp.int32),
            pltpu.VMEM((CH, _LS), jnp.float32),
            pltpu.SemaphoreType.DMA,
        ],
    )
    def k(table_hbm, idx_hbm, out_hbm, idx_v, rows_v, sem):
        wid = lax.axis_index("s") * NC + lax.axis_index("c")
        base = wid * b_per_w
        for g in range(b_per_w // CH):
            pltpu.sync_copy(idx_hbm.at[pl.ds(base + g * CH, CH)], idx_v)
            pltpu.async_copy(table_hbm.at[idx_v], rows_v, sem).wait()
            pltpu.sync_copy(rows_v, out_hbm.at[pl.ds(base + g * CH, CH)])

    return k(table, idx_flat)


def kernel(x, W_conv, b_conv, codebook, W1, b1, W2, b2):
    wflat = W_conv.transpose(0, 2, 1).reshape(_C, _S * _C)
    c2 = jnp.sum(codebook * codebook, axis=-1)[None, :]
    cbt = codebook.T
    a = jnp.arange(512, dtype=jnp.int32)
    kk, tt = a // _BLK, a % _BLK
    src = 4 * tt + kk
    T = (a[:, None] == src[None, :]).astype(jnp.bfloat16)

    full = lambda s: pl.BlockSpec(s, lambda b: (0,) * len(s))
    x_de, idx = pl.pallas_call(
        _front_body,
        grid=(_B,),
        in_specs=[
            pl.BlockSpec((1, _C, _L), lambda b: (b, 0, 0)),
            full((512, 512)),
            full((_C, _C * _S)),
            full((_C, 1)),
            full((_LS, _K)),
            full((1, _K)),
        ],
        out_specs=[pl.BlockSpec((1, _C, _LS), lambda b: (b, 0, 0)),
                   pl.BlockSpec((1, _C, 1), lambda b: (b, 0, 0))],
        out_shape=[jax.ShapeDtypeStruct((_B, _C, _LS), jnp.float32),
                   jax.ShapeDtypeStruct((_B, _C, 1), jnp.int32)],
        compiler_params=pltpu.CompilerParams(
            dimension_semantics=("arbitrary",),
        ),
    )(x, T, wflat, b_conv[:, None], cbt, c2)

    q = _sc_gather(codebook, idx.reshape(_B * _C))
    q = q.reshape(_B, _C, _LS)

    out = pl.pallas_call(
        _back_body,
        grid=(_B,),
        in_specs=[
            pl.BlockSpec((1, _C, _LS), lambda b: (b, 0, 0)),
            pl.BlockSpec((1, _C, _LS), lambda b: (b, 0, 0)),
            full((_C, _C)),
            full((_C, 1)),
            full((_C, _C)),
            full((_C, 1)),
        ],
        out_specs=pl.BlockSpec((1, _C, _LS), lambda b: (b, 0, 0)),
        out_shape=jax.ShapeDtypeStruct((_B, _C, _LS), jnp.float32),
        compiler_params=pltpu.CompilerParams(
            dimension_semantics=("arbitrary",),
        ),
    )(x_de, q, W1, b1[:, None], W2, b2[:, None])
    return out


# speedup vs baseline: 1.9043x; 1.9043x over previous
import jax
import jax.numpy as jnp
import numpy as np
from jax.experimental import pallas as pl
from jax.experimental.pallas import tpu as pltpu

_B, _C, _L = 16, 512, 2048
_S = 4
_LS = _L // _S
_K = _LS
_BLK = 128

_DEF = jax.lax.Precision.DEFAULT


def _dot(a, b):
    return jax.lax.dot_general(a, b, (((1,), (0,)), ((), ())),
                               precision=_DEF,
                               preferred_element_type=jnp.float32)


def _vq_body(x_ref, t_ref, wflat_ref, bconv_ref, cb_ref, cbt_ref, c2_ref,
             w1_ref, b1_ref, w2_ref, b2_ref, out_ref):
    T = t_ref[...]
    for bi in range(x_ref.shape[0]):
        xr = x_ref[bi].astype(jnp.bfloat16)
        ys = [_dot(xr[:, 512 * m:512 * (m + 1)], T) for m in range(4)]
        xks = [jnp.concatenate([ys[m][:, k * _BLK:(k + 1) * _BLK]
                                for m in range(4)], axis=1) for k in range(_S)]
        xcol = jnp.concatenate(xks, axis=0)
        x_de = _dot(wflat_ref[...], xcol) + bconv_ref[...]
        x2 = jnp.sum(x_de * x_de, axis=1, keepdims=True)
        scores = _dot(x_de, cbt_ref[...])
        d2 = x2 + c2_ref[...] - 2.0 * scores
        m = jnp.min(d2, axis=1, keepdims=True)
        iota = jax.lax.broadcasted_iota(jnp.int32, (_C, _K), 1)
        idx = jnp.min(jnp.where(d2 == m, iota, _K), axis=1, keepdims=True)
        onehot = (iota == idx).astype(jnp.float32)
        q = _dot(onehot, cb_ref[...])
        t = x_de - q
        h = jnp.maximum(_dot(w1_ref[...], t) + b1_ref[...], 0.0)
        z = _dot(w2_ref[...], h) + b2_ref[...]
        out_ref[bi] = z + q


def kernel(x, W_conv, b_conv, codebook, W1, b1, W2, b2):
    wflat = W_conv.transpose(0, 2, 1).reshape(_C, _S * _C)
    c2 = jnp.sum(codebook * codebook, axis=-1)[None, :]
    cbt = codebook.T
    a = jnp.arange(512, dtype=jnp.int32)
    kk, tt = a // _BLK, a % _BLK
    src = 4 * tt + kk
    T = (a[:, None] == src[None, :]).astype(jnp.bfloat16)

    full = lambda s: pl.BlockSpec(s, lambda b: (0,) * len(s))
    out = pl.pallas_call(
        _vq_body,
        grid=(_B // 2,),
        in_specs=[
            pl.BlockSpec((2, _C, _L), lambda b: (b, 0, 0)),
            full((512, 512)),
            full((_C, _C * _S)),
            full((_C, 1)),
            full((_K, _LS)),
            full((_LS, _K)),
            full((1, _K)),
            full((_C, _C)),
            full((_C, 1)),
            full((_C, _C)),
            full((_C, 1)),
        ],
        out_specs=pl.BlockSpec((2, _C, _LS), lambda b: (b, 0, 0)),
        out_shape=jax.ShapeDtypeStruct((_B, _C, _LS), jnp.float32),
        compiler_params=pltpu.CompilerParams(
            dimension_semantics=("arbitrary",),
        ),
    )(x, T, wflat, b_conv[:, None], codebook, cbt, c2,
      W1, b1[:, None], W2, b2[:, None])
    return out
